# baseline (speedup 1.0000x reference)
import functools

import jax
import jax.numpy as jnp
from jax import lax
from jax.experimental import pallas as pl
from jax.experimental.pallas import tpu as pltpu
from jax.experimental.pallas import tpu_sc as plsc

B = 1024
D = 8192
BS = 512
NC = 2
NS = 16
L = 16
NW = NC * NS
BPW = BS // NW
NROUND = BPW
NBUF = 4
NVEC = 8

_mesh = plsc.VectorSubcoreMesh(
    core_axis_name="c", subcore_axis_name="s", num_cores=NC, num_subcores=NS)


@functools.partial(
    pl.kernel,
    out_type=jax.ShapeDtypeStruct((NW, L), jnp.float32),
    mesh=_mesh,
    scratch_types=[
        pltpu.VMEM((NROUND, 1), jnp.int32),
        pltpu.VMEM((NBUF, 1, D), jnp.float32),
        pltpu.VMEM((NBUF, 1, D), jnp.float32),
        pltpu.VMEM((L,), jnp.float32),
        pltpu.SemaphoreType.DMA((NBUF,)),
        pltpu.SemaphoreType.DMA((NBUF,)),
    ],
)
def _center_loss_partials(feat_hbm, lab_hbm, cent_hbm, out_hbm,
                          idx_v, fbuf, cbuf, accv, fsems, csems):
    wid = lax.axis_index("s") * NC + lax.axis_index("c")
    base = wid * BPW
    pltpu.sync_copy(lab_hbm.at[wid], idx_v)

    def start(g, b):
        pltpu.make_async_copy(
            feat_hbm.at[pl.ds(base + g, 1)], fbuf.at[b], fsems.at[b]).start()
        pltpu.make_async_copy(
            cent_hbm.at[idx_v.at[g]], cbuf.at[b], csems.at[b]).start()

    def wait(b):
        pltpu.make_async_copy(
            feat_hbm.at[pl.ds(0, 1)], fbuf.at[b], fsems.at[b]).wait()
        pltpu.make_async_copy(
            cent_hbm.at[idx_v.at[0]], cbuf.at[b], csems.at[b]).wait()

    def compute(b, accs):
        def vbody(i, accs):
            f = [fbuf[b, 0, pl.ds(i + j * L, L)] for j in range(NVEC)]
            c = [cbuf[b, 0, pl.ds(i + j * L, L)] for j in range(NVEC)]
            d = [f[j] - c[j] for j in range(NVEC)]
            return tuple(accs[j] + d[j] * d[j] for j in range(NVEC))
        return plsc.parallel_loop(0, D, step=NVEC * L, carry=accs)(vbody)

    for b in range(NBUF):
        start(b, b)

    def outer(t, accs):
        for b in range(NBUF):
            g = t * NBUF + b
            wait(b)
            accs = compute(b, accs)

            @pl.when(g + NBUF < NROUND)
            def _():
                start(g + NBUF, b)
        return accs

    zero = jnp.zeros((L,), jnp.float32)
    accs = lax.fori_loop(0, NROUND // NBUF, outer, (zero,) * NVEC)
    acc = accs[0]
    for j in range(1, NVEC):
        acc = acc + accs[j]
    accv[...] = acc
    pltpu.sync_copy(accv, out_hbm.at[wid])


def _tc_body(lab_ref, f_ref, c_ref, o_ref):
    i = pl.program_id(0)

    @pl.when(i == 0)
    def _():
        o_ref[0, 0] = jnp.float32(0.0)

    d = f_ref[0] - c_ref[0]
    o_ref[0, 0] += jnp.sum(d * d)


_tc_tail = pl.pallas_call(
    _tc_body,
    grid_spec=pltpu.PrefetchScalarGridSpec(
        num_scalar_prefetch=1,
        grid=(B - BS,),
        in_specs=[
            pl.BlockSpec((1, 1, D), lambda i, lab: (i + BS, 0, 0)),
            pl.BlockSpec((1, 1, D), lambda i, lab: (lab[i + BS], 0, 0)),
        ],
        out_specs=pl.BlockSpec((1, 1), lambda i, lab: (0, 0),
                               memory_space=pltpu.SMEM),
    ),
    out_shape=jax.ShapeDtypeStruct((1, 1), jnp.float32),
)


def kernel(features, labels, centers):
    lab = labels.astype(jnp.int32)
    lab_sc = lab[:BS].reshape(NW, BPW, 1)
    partials = _center_loss_partials(features, lab_sc, centers)
    tail = _tc_tail(lab, features.reshape(B, 1, D),
                    centers.reshape(centers.shape[0], 1, D))
    return 0.5 * (jnp.sum(partials) + tail[0, 0]) / features.shape[0]


# speedup vs baseline: 1.0948x; 1.0948x over previous
import functools

import jax
import jax.numpy as jnp
from jax import lax
from jax.experimental import pallas as pl
from jax.experimental.pallas import tpu as pltpu
from jax.experimental.pallas import tpu_sc as plsc

B = 1024
D = 8192
BS = 512
NC = 2
NS = 16
L = 16
NW = NC * NS
BPW = BS // NW
NROUND = BPW
NBUF = 4
NVEC = 8

_mesh = plsc.VectorSubcoreMesh(
    core_axis_name="c", subcore_axis_name="s", num_cores=NC, num_subcores=NS)


@functools.partial(
    pl.kernel,
    out_type=jax.ShapeDtypeStruct((NW, L), jnp.float32),
    mesh=_mesh,
    scratch_types=[
        pltpu.VMEM((NROUND, 1), jnp.int32),
        pltpu.VMEM((NBUF, 1, D), jnp.float32),
        pltpu.VMEM((NBUF, 1, D), jnp.float32),
        pltpu.VMEM((L,), jnp.float32),
        pltpu.SemaphoreType.DMA((NBUF,)),
        pltpu.SemaphoreType.DMA((NBUF,)),
    ],
)
def _center_loss_partials(feat_hbm, lab_hbm, cent_hbm, out_hbm,
                          idx_v, fbuf, cbuf, accv, fsems, csems):
    wid = lax.axis_index("s") * NC + lax.axis_index("c")
    base = wid * BPW
    pltpu.sync_copy(lab_hbm.at[wid], idx_v)

    def start(g, b):
        pltpu.make_async_copy(
            feat_hbm.at[pl.ds(base + g, 1)], fbuf.at[b], fsems.at[b]).start()
        pltpu.make_async_copy(
            cent_hbm.at[idx_v.at[g]], cbuf.at[b], csems.at[b]).start()

    def wait(b):
        pltpu.make_async_copy(
            feat_hbm.at[pl.ds(0, 1)], fbuf.at[b], fsems.at[b]).wait()
        pltpu.make_async_copy(
            cent_hbm.at[idx_v.at[0]], cbuf.at[b], csems.at[b]).wait()

    def compute(b, accs):
        def vbody(i, accs):
            f = [fbuf[b, 0, pl.ds(i + j * L, L)] for j in range(NVEC)]
            c = [cbuf[b, 0, pl.ds(i + j * L, L)] for j in range(NVEC)]
            d = [f[j] - c[j] for j in range(NVEC)]
            return tuple(accs[j] + d[j] * d[j] for j in range(NVEC))
        return plsc.parallel_loop(0, D, step=NVEC * L, carry=accs)(vbody)

    for b in range(NBUF):
        start(b, b)

    def outer(t, accs):
        for b in range(NBUF):
            g = t * NBUF + b
            wait(b)
            accs = compute(b, accs)

            @pl.when(g + NBUF < NROUND)
            def _():
                start(g + NBUF, b)
        return accs

    zero = jnp.zeros((L,), jnp.float32)
    accs = lax.fori_loop(0, NROUND // NBUF, outer, (zero,) * NVEC)
    acc = accs[0]
    for j in range(1, NVEC):
        acc = acc + accs[j]
    accv[...] = acc
    pltpu.sync_copy(accv, out_hbm.at[wid])


def _tc_body(lab_ref, f_ref, c_ref, o_ref):
    i = pl.program_id(0)

    @pl.when(i == 0)
    def _():
        o_ref[0, 0] = jnp.float32(0.0)

    d = f_ref[0] - c_ref[0]
    o_ref[0, 0] += jnp.sum(d * d)


_tc_tail = pl.pallas_call(
    _tc_body,
    grid_spec=pltpu.PrefetchScalarGridSpec(
        num_scalar_prefetch=1,
        grid=(B - BS,),
        in_specs=[
            pl.BlockSpec((1, 8, D // 8), lambda i, lab: (i + BS, 0, 0)),
            pl.BlockSpec((1, 8, D // 8), lambda i, lab: (lab[i + BS], 0, 0)),
        ],
        out_specs=pl.BlockSpec((1, 1), lambda i, lab: (0, 0),
                               memory_space=pltpu.SMEM),
    ),
    out_shape=jax.ShapeDtypeStruct((1, 1), jnp.float32),
)


def kernel(features, labels, centers):
    lab = labels.astype(jnp.int32)
    lab_sc = lab[:BS].reshape(NW, BPW, 1)
    partials = _center_loss_partials(features, lab_sc, centers)
    tail = _tc_tail(lab, features.reshape(B, 8, D // 8),
                    centers.reshape(centers.shape[0], 8, D // 8))
    return 0.5 * (jnp.sum(partials) + tail[0, 0]) / features.shape[0]


# speedup vs baseline: 6.0076x; 5.4873x over previous
import functools

import jax
import jax.numpy as jnp
from jax import lax
from jax.experimental import pallas as pl
from jax.experimental.pallas import tpu as pltpu
from jax.experimental.pallas import tpu_sc as plsc

B = 1024
D = 8192
BS = 512
NC = 2
NS = 16
L = 16
NW = NC * NS
BPW = BS // NW
NROUND = BPW
NBUF = 4
NVEC = 8

_mesh = plsc.VectorSubcoreMesh(
    core_axis_name="c", subcore_axis_name="s", num_cores=NC, num_subcores=NS)


@functools.partial(
    pl.kernel,
    out_type=jax.ShapeDtypeStruct((NW, L), jnp.float32),
    mesh=_mesh,
    scratch_types=[
        pltpu.VMEM((NROUND, 1), jnp.int32),
        pltpu.VMEM((NBUF, 1, D), jnp.float32),
        pltpu.VMEM((NBUF, 1, D), jnp.float32),
        pltpu.VMEM((L,), jnp.float32),
        pltpu.SemaphoreType.DMA((NBUF,)),
        pltpu.SemaphoreType.DMA((NBUF,)),
    ],
)
def _center_loss_partials(feat_hbm, lab_hbm, cent_hbm, out_hbm,
                          idx_v, fbuf, cbuf, accv, fsems, csems):
    wid = lax.axis_index("s") * NC + lax.axis_index("c")
    base = wid * BPW
    pltpu.sync_copy(lab_hbm.at[wid], idx_v)

    def start(g, b):
        pltpu.make_async_copy(
            feat_hbm.at[pl.ds(base + g, 1)], fbuf.at[b], fsems.at[b]).start()
        pltpu.make_async_copy(
            cent_hbm.at[idx_v.at[g]], cbuf.at[b], csems.at[b]).start()

    def wait(b):
        pltpu.make_async_copy(
            feat_hbm.at[pl.ds(0, 1)], fbuf.at[b], fsems.at[b]).wait()
        pltpu.make_async_copy(
            cent_hbm.at[idx_v.at[0]], cbuf.at[b], csems.at[b]).wait()

    def compute(b, accs):
        def vbody(i, accs):
            f = [fbuf[b, 0, pl.ds(i + j * L, L)] for j in range(NVEC)]
            c = [cbuf[b, 0, pl.ds(i + j * L, L)] for j in range(NVEC)]
            d = [f[j] - c[j] for j in range(NVEC)]
            return tuple(accs[j] + d[j] * d[j] for j in range(NVEC))
        return plsc.parallel_loop(0, D, step=NVEC * L, carry=accs)(vbody)

    for b in range(NBUF):
        start(b, b)

    def outer(t, accs):
        for b in range(NBUF):
            g = t * NBUF + b
            wait(b)
            accs = compute(b, accs)

            @pl.when(g + NBUF < NROUND)
            def _():
                start(g + NBUF, b)
        return accs

    zero = jnp.zeros((L,), jnp.float32)
    accs = lax.fori_loop(0, NROUND // NBUF, outer, (zero,) * NVEC)
    acc = accs[0]
    for j in range(1, NVEC):
        acc = acc + accs[j]
    accv[...] = acc
    pltpu.sync_copy(accv, out_hbm.at[wid])


GR = 8
NGRP = (B - BS) // GR
CCH = 1024


def _tc_body(lab_ref, f_hbm, c_hbm, o_ref, fbuf, cbuf, fsems, csems):
    def start(t, s):
        pltpu.make_async_copy(
            f_hbm.at[pl.ds(BS + t * GR, GR)], fbuf.at[s], fsems.at[s]).start()
        for r in range(GR):
            pltpu.make_async_copy(
                c_hbm.at[lab_ref[BS + t * GR + r]], cbuf.at[s, r],
                csems.at[s]).start()

    def wait(s):
        pltpu.make_async_copy(
            f_hbm.at[pl.ds(0, GR)], fbuf.at[s], fsems.at[s]).wait()
        for r in range(GR):
            pltpu.make_async_copy(
                c_hbm.at[0], cbuf.at[s, r], csems.at[s]).wait()

    start(0, 0)
    start(1, 1)

    def outer(tt, acc):
        for s in range(2):
            t = tt * 2 + s
            wait(s)
            for k in range(D // CCH):
                fk = fbuf[s, :, pl.ds(k * CCH, CCH)]
                ck = cbuf[s, :, pl.ds(k * CCH, CCH)]
                d = fk - ck
                acc = acc + d * d

            @pl.when(t + 2 < NGRP)
            def _():
                start(t + 2, s)
        return acc

    acc = lax.fori_loop(0, NGRP // 2, outer,
                        jnp.zeros((GR, CCH), jnp.float32))
    o_ref[0, 0] = jnp.sum(acc)


_tc_tail = pl.pallas_call(
    _tc_body,
    grid_spec=pltpu.PrefetchScalarGridSpec(
        num_scalar_prefetch=1,
        grid=(1,),
        in_specs=[
            pl.BlockSpec(memory_space=pl.ANY),
            pl.BlockSpec(memory_space=pl.ANY),
        ],
        out_specs=pl.BlockSpec(memory_space=pltpu.SMEM),
        scratch_shapes=[
            pltpu.VMEM((2, GR, D), jnp.float32),
            pltpu.VMEM((2, GR, D), jnp.float32),
            pltpu.SemaphoreType.DMA((2,)),
            pltpu.SemaphoreType.DMA((2,)),
        ],
    ),
    out_shape=jax.ShapeDtypeStruct((1, 1), jnp.float32),
)


def kernel(features, labels, centers):
    lab = labels.astype(jnp.int32)
    lab_sc = lab[:BS].reshape(NW, BPW, 1)
    partials = _center_loss_partials(features, lab_sc, centers)
    tail = _tc_tail(lab, features, centers)
    return 0.5 * (jnp.sum(partials) + tail[0, 0]) / features.shape[0]


# speedup vs baseline: 7.9933x; 1.3305x over previous
import functools

import jax
import jax.numpy as jnp
from jax import lax
from jax.experimental import pallas as pl
from jax.experimental.pallas import tpu as pltpu
from jax.experimental.pallas import tpu_sc as plsc

B = 1024
D = 8192
BS = 768
NC = 2
NS = 16
L = 16
NW = NC * NS
BPW = BS // NW
NROUND = BPW
NBUF = 4
NVEC = 8

_mesh = plsc.VectorSubcoreMesh(
    core_axis_name="c", subcore_axis_name="s", num_cores=NC, num_subcores=NS)


@functools.partial(
    pl.kernel,
    out_type=jax.ShapeDtypeStruct((NW, L), jnp.float32),
    mesh=_mesh,
    scratch_types=[
        pltpu.VMEM((NROUND, 1), jnp.int32),
        pltpu.VMEM((NBUF, 1, D), jnp.float32),
        pltpu.VMEM((NBUF, 1, D), jnp.float32),
        pltpu.VMEM((L,), jnp.float32),
        pltpu.SemaphoreType.DMA((NBUF,)),
        pltpu.SemaphoreType.DMA((NBUF,)),
    ],
)
def _center_loss_partials(feat_hbm, lab_hbm, cent_hbm, out_hbm,
                          idx_v, fbuf, cbuf, accv, fsems, csems):
    wid = lax.axis_index("s") * NC + lax.axis_index("c")
    base = wid * BPW
    pltpu.sync_copy(lab_hbm.at[wid], idx_v)

    def start(g, b):
        pltpu.make_async_copy(
            feat_hbm.at[pl.ds(base + g, 1)], fbuf.at[b], fsems.at[b]).start()
        pltpu.make_async_copy(
            cent_hbm.at[idx_v.at[g]], cbuf.at[b], csems.at[b]).start()

    def wait(b):
        pltpu.make_async_copy(
            feat_hbm.at[pl.ds(0, 1)], fbuf.at[b], fsems.at[b]).wait()
        pltpu.make_async_copy(
            cent_hbm.at[idx_v.at[0]], cbuf.at[b], csems.at[b]).wait()

    def compute(b, accs):
        def vbody(i, accs):
            f = [fbuf[b, 0, pl.ds(i + j * L, L)] for j in range(NVEC)]
            c = [cbuf[b, 0, pl.ds(i + j * L, L)] for j in range(NVEC)]
            d = [f[j] - c[j] for j in range(NVEC)]
            return tuple(accs[j] + d[j] * d[j] for j in range(NVEC))
        return plsc.parallel_loop(0, D, step=NVEC * L, carry=accs)(vbody)

    for b in range(NBUF):
        start(b, b)

    def outer(t, accs):
        for b in range(NBUF):
            g = t * NBUF + b
            wait(b)
            accs = compute(b, accs)

            @pl.when(g + NBUF < NROUND)
            def _():
                start(g + NBUF, b)
        return accs

    zero = jnp.zeros((L,), jnp.float32)
    accs = lax.fori_loop(0, NROUND // NBUF, outer, (zero,) * NVEC)
    acc = accs[0]
    for j in range(1, NVEC):
        acc = acc + accs[j]
    accv[...] = acc
    pltpu.sync_copy(accv, out_hbm.at[wid])


GR = 16
NGRP = (B - BS) // GR
CCH = 1024


def _tc_body(lab_ref, f_hbm, c_hbm, o_ref, fbuf, cbuf, fsems, csems):
    def start(t, s):
        pltpu.make_async_copy(
            f_hbm.at[pl.ds(BS + t * GR, GR)], fbuf.at[s], fsems.at[s]).start()
        for r in range(GR):
            pltpu.make_async_copy(
                c_hbm.at[lab_ref[BS + t * GR + r]], cbuf.at[s, r],
                csems.at[s]).start()

    def wait(s):
        pltpu.make_async_copy(
            f_hbm.at[pl.ds(0, GR)], fbuf.at[s], fsems.at[s]).wait()
        for r in range(GR):
            pltpu.make_async_copy(
                c_hbm.at[0], cbuf.at[s, r], csems.at[s]).wait()

    start(0, 0)
    start(1, 1)

    def outer(tt, acc):
        for s in range(2):
            t = tt * 2 + s
            wait(s)
            for k in range(D // CCH):
                fk = fbuf[s, :, pl.ds(k * CCH, CCH)]
                ck = cbuf[s, :, pl.ds(k * CCH, CCH)]
                d = fk - ck
                acc = acc + d * d

            @pl.when(t + 2 < NGRP)
            def _():
                start(t + 2, s)
        return acc

    acc = lax.fori_loop(0, NGRP // 2, outer,
                        jnp.zeros((GR, CCH), jnp.float32))
    o_ref[0, 0] = jnp.sum(acc)


_tc_tail = pl.pallas_call(
    _tc_body,
    grid_spec=pltpu.PrefetchScalarGridSpec(
        num_scalar_prefetch=1,
        grid=(1,),
        in_specs=[
            pl.BlockSpec(memory_space=pl.ANY),
            pl.BlockSpec(memory_space=pl.ANY),
        ],
        out_specs=pl.BlockSpec(memory_space=pltpu.SMEM),
        scratch_shapes=[
            pltpu.VMEM((2, GR, D), jnp.float32),
            pltpu.VMEM((2, GR, D), jnp.float32),
            pltpu.SemaphoreType.DMA((2,)),
            pltpu.SemaphoreType.DMA((2,)),
        ],
    ),
    out_shape=jax.ShapeDtypeStruct((1, 1), jnp.float32),
)


def kernel(features, labels, centers):
    lab = labels.astype(jnp.int32)
    lab_sc = lab[:BS].reshape(NW, BPW, 1)
    partials = _center_loss_partials(features, lab_sc, centers)
    tail = _tc_tail(lab, features, centers)
    return 0.5 * (jnp.sum(partials) + tail[0, 0]) / features.shape[0]
